# baseline (speedup 1.0000x reference)
import jax
import jax.numpy as jnp
from jax import lax
from jax.experimental import pallas as pl
from jax.experimental.pallas import tpu as pltpu
from jax.experimental.pallas import tpu_sc as plsc

_NUM_TASKS = 8
_NUM_MODALITIES = 4
_PRIOR_LEN = 16
_EMBED_DIM = 768
_BATCH = 4096

_ROW = _PRIOR_LEN * _EMBED_DIM
_NROWS = 2 * _BATCH
_NC, _NS = 2, 16
_NW = _NC * _NS
_ROWS_PER_W = _NROWS // _NW
_CHUNK = 4
_NCHUNK = _ROWS_PER_W // _CHUNK


def _body(table_hbm, idx_hbm, pos_hbm, out_hbm, idx_v, pos_v, buf_a, buf_b,
          gsem_a, gsem_b, ssem_a, ssem_b):
    wid = lax.axis_index("s") * _NC + lax.axis_index("c")
    pltpu.sync_copy(idx_hbm.at[wid], idx_v)
    pltpu.sync_copy(pos_hbm.at[wid], pos_v)

    def _gather(j, buf, sem):
        pltpu.async_copy(table_hbm.at[idx_v.at[j]], buf, sem)

    def _wait_gather(j, buf, sem):
        pltpu.make_async_copy(table_hbm.at[idx_v.at[j]], buf, sem).wait()

    def _scatter(j, buf, sem):
        pltpu.async_copy(buf, out_hbm.at[pos_v.at[j]], sem)

    def _wait_scatter(j, buf, sem):
        pltpu.make_async_copy(buf, out_hbm.at[pos_v.at[j]], sem).wait()

    _gather(0, buf_a, gsem_a)

    @pl.loop(0, _NCHUNK // 2)
    def _pair(i):
        j0 = 2 * i
        j1 = j0 + 1

        @pl.when(i > 0)
        def _():
            _wait_scatter(j0 - 1, buf_b, ssem_b)

        _gather(j1, buf_b, gsem_b)
        _wait_gather(j0, buf_a, gsem_a)
        _scatter(j0, buf_a, ssem_a)
        _wait_gather(j1, buf_b, gsem_b)
        _scatter(j1, buf_b, ssem_b)
        _wait_scatter(j0, buf_a, ssem_a)

        @pl.when(j1 + 1 < _NCHUNK)
        def _():
            _gather(j1 + 1, buf_a, gsem_a)

    _wait_scatter(_NCHUNK - 1, buf_b, ssem_b)


_sc_gather = pl.kernel(
    _body,
    out_type=jax.ShapeDtypeStruct((_NROWS, _ROW), jnp.float32),
    mesh=plsc.VectorSubcoreMesh(
        core_axis_name="c", subcore_axis_name="s",
        num_cores=_NC, num_subcores=_NS,
    ),
    scratch_types=[
        pltpu.VMEM((_NCHUNK, _CHUNK), jnp.int32),
        pltpu.VMEM((_NCHUNK, _CHUNK), jnp.int32),
        pltpu.VMEM((_CHUNK, _ROW), jnp.float32),
        pltpu.VMEM((_CHUNK, _ROW), jnp.float32),
        pltpu.SemaphoreType.DMA,
        pltpu.SemaphoreType.DMA,
        pltpu.SemaphoreType.DMA,
        pltpu.SemaphoreType.DMA,
    ],
)


def kernel(task_table, modality_table, task_idx, modality_idx):
    table = jnp.concatenate(
        [task_table.reshape(_NUM_TASKS, _ROW),
         modality_table.reshape(_NUM_MODALITIES, _ROW)], axis=0)
    idx = jnp.stack(
        [task_idx.astype(jnp.int32),
         modality_idx.astype(jnp.int32) + _NUM_TASKS], axis=1)
    idx = idx.reshape(_NW, _NCHUNK, _CHUNK)
    pos = jnp.arange(_NROWS, dtype=jnp.int32).reshape(_NW, _NCHUNK, _CHUNK)
    out = _sc_gather(table, idx, pos)
    return out.reshape(_BATCH, 2 * _PRIOR_LEN, _EMBED_DIM)
